# Initial kernel scaffold; baseline (speedup 1.0000x reference)
import jax
import jax.numpy as jnp
from jax.experimental import pallas as pl


def kernel(x, edge_index, edge_weight, W):
    raise NotImplementedError("write your pallas kernel here")


# speedup vs baseline: 2.4578x; 2.4578x over previous
import jax
import jax.numpy as jnp
from jax import lax
from jax.experimental import pallas as pl
from jax.experimental.pallas import tpu as pltpu
from jax.experimental.pallas import tpu_sc as plsc

N = 10000
N_PAD = 10240
D = 128
NC = 2
NS = 16
L = 16
EPW = 10240
E_PAD = NC * NS * EPW
C = 256
CH = C // 128
CHUNKS = EPW // C
ROWS_PER_SUB = N_PAD // NS


def _sc_body(x_hbm, src_hbm, dst_hbm, ew_hbm, out_hbm,
             src0, src1, dst0, dst1,
             ew_v, rows_v, acc):
    cid = lax.axis_index("c")
    sid = lax.axis_index("s")
    src_bufs = (src0, src1)
    dst_bufs = (dst0, dst1)

    @plsc.parallel_loop(0, C)
    def _zero(i):
        for j in range(D // L):
            rows_v[i, pl.ds(j * L, L)] = jnp.zeros((L,), jnp.float32)

    base_acc = sid * ROWS_PER_SUB
    off = 0
    while off < ROWS_PER_SUB:
        n = min(C, ROWS_PER_SUB - off)
        pltpu.sync_copy(rows_v.at[pl.ds(0, n)],
                        acc.at[pl.ds(base_acc + off, n)])
        off += n
    plsc.subcore_barrier()

    wid = cid * NS + sid
    flat_base = wid * EPW

    def chunk(k, _):
        e0 = flat_base + k * C
        for j in range(CH):
            pltpu.sync_copy(src_hbm.at[pl.ds(e0 + j * 128, 128)], src_bufs[j])
            pltpu.sync_copy(dst_hbm.at[pl.ds(e0 + j * 128, 128)], dst_bufs[j])
        pltpu.sync_copy(ew_hbm.at[pl.ds(e0, C)], ew_v)

        for j in range(CH):
            pltpu.sync_copy(x_hbm.at[src_bufs[j]],
                            rows_v.at[pl.ds(j * 128, 128)])

        @plsc.parallel_loop(0, C // L)
        def _scale(g):
            wg = ew_v[pl.ds(g * L, L)]
            for r in range(L):
                w = jnp.take_along_axis(wg, jnp.full((L,), r, jnp.int32), 0)
                i = g * L + r
                for j in range(D // L):
                    rows_v[i, pl.ds(j * L, L)] = rows_v[i, pl.ds(j * L, L)] * w

        for j in range(CH):
            pltpu.sync_copy(rows_v.at[pl.ds(j * 128, 128)],
                            acc.at[dst_bufs[j]], add=True)
        return 0

    lax.fori_loop(0, CHUNKS, chunk, 0)

    plsc.subcore_barrier()
    pltpu.sync_copy(acc.at[pl.ds(base_acc, ROWS_PER_SUB)],
                    out_hbm.at[cid, pl.ds(base_acc, ROWS_PER_SUB)])


_sc_agg = pl.kernel(
    _sc_body,
    out_type=jax.ShapeDtypeStruct((NC, N_PAD, D), jnp.float32),
    mesh=plsc.VectorSubcoreMesh(core_axis_name="c", subcore_axis_name="s"),
    scratch_types=(
        [pltpu.VMEM((128,), jnp.int32) for _ in range(2 * CH)]
        + [
            pltpu.VMEM((C,), jnp.float32),
            pltpu.VMEM((C, D), jnp.float32),
            pltpu.VMEM_SHARED((N_PAD, D), jnp.float32),
        ]
    ),
)


def _tc_body(p_ref, w_ref, o_ref):
    s = p_ref[0] + p_ref[1]
    o_ref[...] = jnp.maximum(
        jnp.dot(s, w_ref[...], preferred_element_type=jnp.float32), 0.0)


BLK = 1000


@jax.jit
def _tc_combine(partials, W):
    return pl.pallas_call(
        _tc_body,
        out_shape=jax.ShapeDtypeStruct((N, D), jnp.float32),
        grid=(N // BLK,),
        in_specs=[
            pl.BlockSpec((NC, BLK, D), lambda i: (0, i, 0)),
            pl.BlockSpec((D, D), lambda i: (0, 0)),
        ],
        out_specs=pl.BlockSpec((BLK, D), lambda i: (i, 0)),
    )(partials, W)


@jax.jit
def kernel(x, edge_index, edge_weight, W):
    E = edge_weight.shape[0]
    pad = E_PAD - E
    src = jnp.pad(edge_index[0].astype(jnp.int32), (0, pad))
    dst = jnp.pad(edge_index[1].astype(jnp.int32), (0, pad))
    ew = jnp.pad(edge_weight, (0, pad))
    partials = _sc_agg(x, src, dst, ew)
    return _tc_combine(partials, W)
